# baseline (speedup 1.0000x reference)
import functools

import jax
import jax.numpy as jnp
from jax import lax
from jax.experimental import pallas as pl
from jax.experimental.pallas import tpu as pltpu
from jax.experimental.pallas import tpu_sc as plsc

N = 10000
E = 160000
D_IN = 256
H = 512

NC = 2
NS = 16
BATCH = 128
CBATCH = 96
NPAD = 10112
ROWS_PER_TILE = NPAD // NS
EP = 165888
EDGES_PER_TILE = EP // NS
NBATCH = EDGES_PER_TILE // BATCH
CHUNK = 128
LANES = 16
NG = 3
ND = 3

_mesh = functools.partial(
    plsc.VectorSubcoreMesh, core_axis_name="c", subcore_axis_name="s",
    num_cores=NC, num_subcores=NS)


@functools.lru_cache(maxsize=None)
def _make_agg_kernel(nchunks):
  per_sc = nchunks // NC

  def body(src1d, dst1d, zeros, h_flat, out_flat, *rest):
    rows = rest[:NG]
    gsems = rest[NG:2 * NG]
    sbufs = rest[2 * NG:2 * NG + ND]
    isems = rest[2 * NG + ND:2 * NG + 2 * ND]
    dbufs = rest[2 * NG + 2 * ND:2 * NG + 3 * ND]
    dsems = rest[2 * NG + 3 * ND:2 * NG + 4 * ND]
    acc = rest[2 * NG + 4 * ND]

    cid = lax.axis_index("c")
    sid = lax.axis_index("s")
    ebase = pl.multiple_of(sid * EDGES_PER_TILE, BATCH)
    rbase = pl.multiple_of(sid * ROWS_PER_TILE, 8)

    def eslice(i):
      return pl.ds(pl.multiple_of(ebase + i * BATCH, BATCH), BATCH)

    def bias_and_gather(sb, delta, r):
      for q in range(BATCH // LANES):
        sl = pl.ds(q * LANES, LANES)
        sb[sl] = sb[sl] + delta
      pltpu.async_copy(h_flat.at[sb], rows[r], gsems[r])

    for lc in range(per_sc):
      ch = cid * per_sc + lc
      delta = ch * NPAD

      pltpu.sync_copy(zeros.at[pl.ds(rbase, ROWS_PER_TILE)],
                      acc.at[pl.ds(rbase, ROWS_PER_TILE)])
      plsc.subcore_barrier()

      for q in range(ND):
        pltpu.async_copy(src1d.at[eslice(q)], sbufs[q], isems[q])
        pltpu.async_copy(dst1d.at[eslice(q)], dbufs[q], dsems[q])
      for j in range(NG):
        pltpu.make_async_copy(
            src1d.at[pl.ds(0, BATCH)], sbufs[j], isems[j]).wait()
        bias_and_gather(sbufs[j], delta, j)

      @pl.loop(0, NBATCH, step=ND)
      def _(b0):
        for j in range(ND):
          i = b0 + j
          pltpu.make_async_copy(
              h_flat.at[sbufs[j]], rows[j], gsems[j]).wait()

          @pl.when(i + ND < NBATCH)
          def _():
            pltpu.async_copy(src1d.at[eslice(i + ND)], sbufs[j], isems[j])

          pltpu.make_async_copy(
              dst1d.at[pl.ds(0, BATCH)], dbufs[j], dsems[j]).wait()
          pltpu.sync_copy(rows[j], acc.at[dbufs[j]], add=True)

          @pl.when(i + ND < NBATCH)
          def _():
            pltpu.async_copy(dst1d.at[eslice(i + ND)], dbufs[j], dsems[j])
            pltpu.make_async_copy(
                src1d.at[pl.ds(0, BATCH)], sbufs[j], isems[j]).wait()
            bias_and_gather(sbufs[j], delta, j)

      plsc.subcore_barrier()
      obase = pl.multiple_of(ch * NPAD + rbase, 8)
      pltpu.sync_copy(acc.at[pl.ds(rbase, ROWS_PER_TILE)],
                      out_flat.at[pl.ds(obase, ROWS_PER_TILE)])

  return pl.kernel(
      body,
      out_type=jax.ShapeDtypeStruct((nchunks * NPAD, CHUNK), jnp.float32),
      mesh=_mesh(),
      scratch_types=(
          [pltpu.VMEM((BATCH, CHUNK), jnp.float32)] * NG
          + [pltpu.SemaphoreType.DMA] * NG
          + [pltpu.VMEM((BATCH,), jnp.int32)] * ND
          + [pltpu.SemaphoreType.DMA] * ND
          + [pltpu.VMEM((BATCH,), jnp.int32)] * ND
          + [pltpu.SemaphoreType.DMA] * ND
          + [pltpu.VMEM_SHARED((NPAD, CHUNK), jnp.float32)]
      ),
      name=f"sc_segment_sum_{nchunks}",
  )


def _count_kernel_body(dst1d, zeros, ones, cnt_flat, ones_v, *rest):
  NDC = 6
  dbufs = rest[:NDC]
  dsems = rest[NDC:2 * NDC]
  acc = rest[2 * NDC]

  cid = lax.axis_index("c")
  sid = lax.axis_index("s")
  nb = EP // NC // NS // CBATCH
  ebase = pl.multiple_of(cid * (EP // NC) + sid * (EP // NC // NS), 8)
  rbase = pl.multiple_of(sid * ROWS_PER_TILE, 8)

  pltpu.sync_copy(ones, ones_v)
  pltpu.sync_copy(zeros.at[pl.ds(rbase, ROWS_PER_TILE)],
                  acc.at[pl.ds(rbase, ROWS_PER_TILE)])
  plsc.subcore_barrier()

  for q in range(NDC):
    pltpu.async_copy(
        dst1d.at[pl.ds(pl.multiple_of(ebase + q * CBATCH, 8), CBATCH)],
        dbufs[q], dsems[q])

  @pl.loop(0, nb, step=NDC)
  def _(b0):
    for j in range(NDC):
      b = b0 + j
      pltpu.make_async_copy(
          dst1d.at[pl.ds(0, CBATCH)], dbufs[j], dsems[j]).wait()
      pltpu.sync_copy(ones_v, acc.at[dbufs[j]], add=True)

      @pl.when(b + NDC < nb)
      def _():
        s = pl.multiple_of(ebase + (b + NDC) * CBATCH, 8)
        pltpu.async_copy(dst1d.at[pl.ds(s, CBATCH)], dbufs[j], dsems[j])

  plsc.subcore_barrier()
  obase = pl.multiple_of(cid * NPAD + rbase, 8)
  pltpu.sync_copy(acc.at[pl.ds(rbase, ROWS_PER_TILE)],
                  cnt_flat.at[pl.ds(obase, ROWS_PER_TILE)])


@functools.lru_cache(maxsize=None)
def _make_count_kernel():
  NDC = 6
  return pl.kernel(
      _count_kernel_body,
      out_type=jax.ShapeDtypeStruct((NC * NPAD, CHUNK), jnp.float32),
      mesh=_mesh(),
      scratch_types=(
          [pltpu.VMEM((CBATCH, CHUNK), jnp.float32)]
          + [pltpu.VMEM((CBATCH,), jnp.int32)] * NDC
          + [pltpu.SemaphoreType.DMA] * NDC
          + [pltpu.VMEM_SHARED((NPAD, CHUNK), jnp.float32)]
      ),
      name="sc_degree_count",
  )


def _dense_layer(agg, h, cnt, Wl, Wr, b, final):
  nch = h.shape[0]
  rows = 1000
  grid = (N // rows,)

  def body(agg_ref, h_ref, cnt_ref, wl, wr, bb, out_ref):
    c = cnt_ref[0, :, :1] + cnt_ref[1, :, :1]
    inv = 1.0 / jnp.maximum(c, 1.0)
    acc = jnp.zeros((rows, H), jnp.float32)
    for k in range(nch):
      acc += jnp.dot(agg_ref[k] * inv, wl[k * CHUNK:(k + 1) * CHUNK, :],
                     preferred_element_type=jnp.float32)
      acc += jnp.dot(h_ref[k], wr[k * CHUNK:(k + 1) * CHUNK, :],
                     preferred_element_type=jnp.float32)
    out = acc + bb[...]
    n2 = jnp.sum(out * out, axis=1, keepdims=True)
    out = out * lax.rsqrt(jnp.maximum(n2, 1e-24))
    out = jnp.maximum(out, 0.0)
    if final:
      out_ref[...] = out
    else:
      for k in range(H // CHUNK):
        out_ref[k] = out[:, k * CHUNK:(k + 1) * CHUNK]

  din = nch * CHUNK
  stk = lambda n: pl.BlockSpec((n, rows, CHUNK), lambda i: (0, i, 0))
  in_specs = [
      stk(nch), stk(nch), stk(NC),
      pl.BlockSpec((din, H), lambda i: (0, 0)),
      pl.BlockSpec((din, H), lambda i: (0, 0)),
      pl.BlockSpec((1, H), lambda i: (0, 0)),
  ]
  if final:
    out_specs = pl.BlockSpec((rows, H), lambda i: (i, 0))
    out_shape = jax.ShapeDtypeStruct((N, H), jnp.float32)
  else:
    out_specs = stk(H // CHUNK)
    out_shape = jax.ShapeDtypeStruct((H // CHUNK, NPAD, CHUNK), jnp.float32)
  return pl.pallas_call(
      body, grid=grid, in_specs=in_specs, out_specs=out_specs,
      out_shape=out_shape,
  )(agg, h, cnt, Wl, Wr, b.reshape(1, H))


@jax.jit
def kernel(x, edge_index, Wl0, Wr0, b0, Wl1, Wr1, b1, Wl2, Wr2, b2):
  src = edge_index[0].astype(jnp.int32)
  dst = edge_index[1].astype(jnp.int32)
  pad = jnp.full((EP - E,), N, jnp.int32)
  src1d = jnp.concatenate([src, pad])
  dst1d = jnp.concatenate([dst, pad])

  zeros = jnp.zeros((NPAD, CHUNK), jnp.float32)
  ones = jnp.ones((CBATCH, CHUNK), jnp.float32)

  cnt = _make_count_kernel()(dst1d, zeros, ones).reshape(NC, NPAD, CHUNK)

  xp = jnp.pad(x, ((0, NPAD - N), (0, 0)))
  h = jnp.stack([xp[:, k * CHUNK:(k + 1) * CHUNK]
                 for k in range(D_IN // CHUNK)])

  agg = _make_agg_kernel(2)(src1d, dst1d, zeros, h.reshape(-1, CHUNK))
  h = _dense_layer(agg.reshape(2, NPAD, CHUNK), h, cnt, Wl0, Wr0, b0,
                   final=False)

  agg = _make_agg_kernel(4)(src1d, dst1d, zeros, h.reshape(-1, CHUNK))
  h = _dense_layer(agg.reshape(4, NPAD, CHUNK), h, cnt, Wl1, Wr1, b1,
                   final=False)

  agg = _make_agg_kernel(4)(src1d, dst1d, zeros, h.reshape(-1, CHUNK))
  return _dense_layer(agg.reshape(4, NPAD, CHUNK), h, cnt, Wl2, Wr2, b2,
                      final=True)


# speedup vs baseline: 1.2312x; 1.2312x over previous
import functools

import jax
import jax.numpy as jnp
from jax import lax
from jax.experimental import pallas as pl
from jax.experimental.pallas import tpu as pltpu
from jax.experimental.pallas import tpu_sc as plsc

N = 10000
E = 160000
D_IN = 256
H = 512

NC = 2
NS = 16
BATCH = 128
NPAD = 10112
ROWS_PER_TILE = NPAD // NS
EP = 163840
EDGES_PER_TILE = EP // NS
NBATCH = EDGES_PER_TILE // BATCH
CHUNK = 128
LANES = 16

_mesh = functools.partial(
    plsc.VectorSubcoreMesh, core_axis_name="c", subcore_axis_name="s",
    num_cores=NC, num_subcores=NS)


@functools.lru_cache(maxsize=None)
def _make_agg_kernel(nchunks):
  per_sc = nchunks // NC
  NG = 2
  ND = 4

  def body(src2d, dst1d, zeros, h_flat, out_flat, src_v, *rest):
    rows = rest[:NG]
    gsems = rest[NG:2 * NG]
    dbufs = rest[2 * NG:2 * NG + ND]
    dsems = rest[2 * NG + ND:2 * NG + 2 * ND]
    acc = rest[2 * NG + 2 * ND]

    cid = lax.axis_index("c")
    sid = lax.axis_index("s")
    ibase = pl.multiple_of(sid * NBATCH, 8)
    ebase = pl.multiple_of(sid * EDGES_PER_TILE, BATCH)
    rbase = pl.multiple_of(sid * ROWS_PER_TILE, 8)

    pltpu.sync_copy(src2d.at[pl.ds(ibase, NBATCH)], src_v)

    for lc in range(per_sc):
      ch = cid * per_sc + lc

      delta = cid * (per_sc * NPAD) if lc == 0 else NPAD

      @pl.loop(0, NBATCH)
      def _(bb):
        for q in range(BATCH // LANES):
          sl = pl.ds(q * LANES, LANES)
          src_v[bb, sl] = src_v[bb, sl] + delta

      pltpu.sync_copy(zeros.at[pl.ds(rbase, ROWS_PER_TILE)],
                      acc.at[pl.ds(rbase, ROWS_PER_TILE)])
      plsc.subcore_barrier()

      for q in range(ND):
        pltpu.async_copy(
            dst1d.at[pl.ds(pl.multiple_of(ebase + q * BATCH, BATCH), BATCH)],
            dbufs[q], dsems[q])
      for j in range(NG):
        pltpu.async_copy(h_flat.at[src_v.at[j]], rows[j], gsems[j])

      @pl.loop(0, NBATCH, step=ND)
      def _(b0):
        for j in range(ND):
          b = b0 + j
          r = j % NG
          pltpu.make_async_copy(
              h_flat.at[src_v.at[j]], rows[r], gsems[r]).wait()
          pltpu.make_async_copy(
              dst1d.at[pl.ds(0, BATCH)], dbufs[j], dsems[j]).wait()
          pltpu.sync_copy(rows[r], acc.at[dbufs[j]], add=True)

          @pl.when(b + NG < NBATCH)
          def _():
            pltpu.async_copy(h_flat.at[src_v.at[b + NG]], rows[r], gsems[r])

          @pl.when(b + ND < NBATCH)
          def _():
            s = pl.multiple_of(ebase + (b + ND) * BATCH, BATCH)
            pltpu.async_copy(dst1d.at[pl.ds(s, BATCH)], dbufs[j], dsems[j])

      plsc.subcore_barrier()
      obase = pl.multiple_of(ch * NPAD + rbase, 8)
      pltpu.sync_copy(acc.at[pl.ds(rbase, ROWS_PER_TILE)],
                      out_flat.at[pl.ds(obase, ROWS_PER_TILE)])

  return pl.kernel(
      body,
      out_type=jax.ShapeDtypeStruct((nchunks * NPAD, CHUNK), jnp.float32),
      mesh=_mesh(),
      scratch_types=(
          [pltpu.VMEM((NBATCH, BATCH), jnp.int32)]
          + [pltpu.VMEM((BATCH, CHUNK), jnp.float32)] * NG
          + [pltpu.SemaphoreType.DMA] * NG
          + [pltpu.VMEM((BATCH,), jnp.int32)] * ND
          + [pltpu.SemaphoreType.DMA] * ND
          + [pltpu.VMEM_SHARED((NPAD, CHUNK), jnp.float32)]
      ),
      name=f"sc_segment_sum_{nchunks}",
  )


def _count_kernel_body(dst1d, zeros, ones, cnt_flat, ones_v, *rest):
  ND = 4
  dbufs = rest[:ND]
  dsems = rest[ND:2 * ND]
  acc = rest[2 * ND]

  cid = lax.axis_index("c")
  sid = lax.axis_index("s")
  nb = EP // NC // NS // BATCH
  ebase = pl.multiple_of(cid * (EP // NC) + sid * (EP // NC // NS), BATCH)
  rbase = pl.multiple_of(sid * ROWS_PER_TILE, 8)

  pltpu.sync_copy(ones, ones_v)
  pltpu.sync_copy(zeros.at[pl.ds(rbase, ROWS_PER_TILE)],
                  acc.at[pl.ds(rbase, ROWS_PER_TILE)])
  plsc.subcore_barrier()

  for q in range(ND):
    pltpu.async_copy(
        dst1d.at[pl.ds(pl.multiple_of(ebase + q * BATCH, BATCH), BATCH)],
        dbufs[q], dsems[q])

  @pl.loop(0, nb, step=ND)
  def _(b0):
    for j in range(ND):
      b = b0 + j
      pltpu.make_async_copy(
          dst1d.at[pl.ds(0, BATCH)], dbufs[j], dsems[j]).wait()
      pltpu.sync_copy(ones_v, acc.at[dbufs[j]], add=True)

      @pl.when(b + ND < nb)
      def _():
        s = pl.multiple_of(ebase + (b + ND) * BATCH, BATCH)
        pltpu.async_copy(dst1d.at[pl.ds(s, BATCH)], dbufs[j], dsems[j])

  plsc.subcore_barrier()
  obase = pl.multiple_of(cid * NPAD + rbase, 8)
  pltpu.sync_copy(acc.at[pl.ds(rbase, ROWS_PER_TILE)],
                  cnt_flat.at[pl.ds(obase, ROWS_PER_TILE)])


@functools.lru_cache(maxsize=None)
def _make_count_kernel():
  ND = 4
  return pl.kernel(
      _count_kernel_body,
      out_type=jax.ShapeDtypeStruct((NC * NPAD, CHUNK), jnp.float32),
      mesh=_mesh(),
      scratch_types=(
          [pltpu.VMEM((BATCH, CHUNK), jnp.float32)]
          + [pltpu.VMEM((BATCH,), jnp.int32)] * ND
          + [pltpu.SemaphoreType.DMA] * ND
          + [pltpu.VMEM_SHARED((NPAD, CHUNK), jnp.float32)]
      ),
      name="sc_degree_count",
  )


def _tc_self(h, Wr, b):
  nch = h.shape[0]
  rows = 1000
  grid = (N // rows,)

  def body(h_ref, wr, bb, out_ref):
    acc = bb[...] + jnp.zeros((rows, H), jnp.float32)
    for k in range(nch):
      acc += jnp.dot(h_ref[k], wr[k * CHUNK:(k + 1) * CHUNK, :],
                     preferred_element_type=jnp.float32)
    out_ref[...] = acc

  din = nch * CHUNK
  return pl.pallas_call(
      body, grid=grid,
      in_specs=[pl.BlockSpec((nch, rows, CHUNK), lambda i: (0, i, 0)),
                pl.BlockSpec((din, H), lambda i: (0, 0)),
                pl.BlockSpec((1, H), lambda i: (0, 0))],
      out_specs=pl.BlockSpec((rows, H), lambda i: (i, 0)),
      out_shape=jax.ShapeDtypeStruct((N, H), jnp.float32),
  )(h, Wr, b.reshape(1, H))


def _tc_combine(agg, slf, cnt, Wl, final):
  nch = agg.shape[0]
  rows = 1000
  grid = (N // rows,)

  def body(agg_ref, slf_ref, cnt_ref, wl, out_ref):
    c = cnt_ref[0, :, :1] + cnt_ref[1, :, :1]
    inv = 1.0 / jnp.maximum(c, 1.0)
    acc = slf_ref[...]
    for k in range(nch):
      acc += jnp.dot(agg_ref[k] * inv, wl[k * CHUNK:(k + 1) * CHUNK, :],
                     preferred_element_type=jnp.float32)
    out = acc
    n2 = jnp.sum(out * out, axis=1, keepdims=True)
    out = out * lax.rsqrt(jnp.maximum(n2, 1e-24))
    out = jnp.maximum(out, 0.0)
    if final:
      out_ref[...] = out
    else:
      for k in range(H // CHUNK):
        out_ref[k] = out[:, k * CHUNK:(k + 1) * CHUNK]

  din = nch * CHUNK
  if final:
    out_specs = pl.BlockSpec((rows, H), lambda i: (i, 0))
    out_shape = jax.ShapeDtypeStruct((N, H), jnp.float32)
  else:
    out_specs = pl.BlockSpec((H // CHUNK, rows, CHUNK), lambda i: (0, i, 0))
    out_shape = jax.ShapeDtypeStruct((H // CHUNK, NPAD, CHUNK), jnp.float32)
  return pl.pallas_call(
      body, grid=grid,
      in_specs=[pl.BlockSpec((nch, rows, CHUNK), lambda i: (0, i, 0)),
                pl.BlockSpec((rows, H), lambda i: (i, 0)),
                pl.BlockSpec((NC, rows, CHUNK), lambda i: (0, i, 0)),
                pl.BlockSpec((din, H), lambda i: (0, 0))],
      out_specs=out_specs, out_shape=out_shape,
  )(agg, slf, cnt, Wl)


@jax.jit
def kernel(x, edge_index, Wl0, Wr0, b0, Wl1, Wr1, b1, Wl2, Wr2, b2):
  src = edge_index[0].astype(jnp.int32)
  dst = edge_index[1].astype(jnp.int32)
  pad = jnp.full((EP - E,), N, jnp.int32)
  src = jnp.concatenate([src, pad]).reshape(EP // BATCH, BATCH)
  dst1d = jnp.concatenate([dst, pad])

  zeros = jnp.zeros((NPAD, CHUNK), jnp.float32)
  ones = jnp.ones((BATCH, CHUNK), jnp.float32)

  cnt = _make_count_kernel()(dst1d, zeros, ones).reshape(NC, NPAD, CHUNK)

  xp = jnp.pad(x, ((0, NPAD - N), (0, 0)))
  h = jnp.stack([xp[:, k * CHUNK:(k + 1) * CHUNK]
                 for k in range(D_IN // CHUNK)])

  for li, (nch, Wl, Wr, b) in enumerate(
      [(2, Wl0, Wr0, b0), (4, Wl1, Wr1, b1), (4, Wl2, Wr2, b2)]):
    final = li == 2
    slf = _tc_self(h, Wr, b)
    agg = _make_agg_kernel(nch)(src, dst1d, zeros, h.reshape(-1, CHUNK))
    h = _tc_combine(agg.reshape(nch, NPAD, CHUNK), slf, cnt, Wl, final)
  return h
